# baseline (speedup 1.0000x reference)
import functools

import jax
import jax.numpy as jnp
from jax import lax
from jax.experimental import pallas as pl
from jax.experimental.pallas import tpu as pltpu
from jax.experimental.pallas import tpu_sc as plsc

_R = 32768
_D = 1024
_CROWS = 2048
_NBUF = 4
_SC_TAIL = 18432
_SC_CHUNK = 32
_SC_NBUF = 3


def _tc_pipeline(chunks, bufs, in_sems, out_sems):
    n = len(chunks)
    ins, outs = [], []
    for j in range(min(_NBUF, n)):
        c = pltpu.make_async_copy(chunks[j][0], bufs.at[j], in_sems.at[j])
        c.start()
        ins.append(c)
    for i in range(n):
        s = i % _NBUF
        ins[i].wait()
        c = pltpu.make_async_copy(bufs.at[s], chunks[i][1], out_sems.at[s])
        c.start()
        outs.append(c)
        ni = i + _NBUF
        if ni < n:
            outs[i].wait()
            c = pltpu.make_async_copy(chunks[ni][0], bufs.at[s], in_sems.at[s])
            c.start()
            ins.append(c)
    for i in range(max(n - _NBUF, 0), n):
        outs[i].wait()


def _tc1_body(pos_ref, x_ref, k_hbm, ox_ref, ok_hbm,
              bufs, row_buf, in_sems, out_sems, row_sem):
    pos = pos_ref[0]
    chunks = [(k_hbm.at[pl.ds(j * _CROWS, _CROWS)],
               ok_hbm.at[pl.ds(j * _CROWS, _CROWS)])
              for j in range(_R // _CROWS)]
    ox_ref[...] = x_ref[...] * x_ref[...]
    row_buf[0, :] = jnp.full((_D,), 100.0, jnp.float32)
    _tc_pipeline(chunks, bufs, in_sems, out_sems)
    c = pltpu.make_async_copy(
        row_buf.at[pl.ds(0, 1)], ok_hbm.at[pl.ds(2048 + pos, 1)], row_sem)
    c.start()
    c.wait()


def _tc2_body(pos_ref, v_hbm, ovp_hbm, ov_hbm,
              bufs, row_buf, in_sems, out_sems, row_sem):
    del ovp_hbm
    pos = pos_ref[0]
    head = _R - _SC_TAIL
    chunks = [(v_hbm.at[pl.ds(j * _CROWS, _CROWS)],
               ov_hbm.at[pl.ds(j * _CROWS, _CROWS)])
              for j in range(head // _CROWS)]
    row_buf[0, :] = jnp.full((_D,), 200.0, jnp.float32)
    _tc_pipeline(chunks, bufs, in_sems, out_sems)
    c = pltpu.make_async_copy(
        row_buf.at[pl.ds(0, 1)], ov_hbm.at[pl.ds(10245 + pos, 1)], row_sem)
    c.start()
    c.wait()


def _tc_call(body, n_in, n_out, operands, aliases):
    grid_spec = pltpu.PrefetchScalarGridSpec(
        num_scalar_prefetch=1,
        grid=(),
        in_specs=[pl.BlockSpec(memory_space=pltpu.VMEM)] * (1 if n_out == 2 else 0)
        + [pl.BlockSpec(memory_space=pl.ANY)] * (n_in - (1 if n_out == 2 else 0)),
        out_specs=[pl.BlockSpec(memory_space=pltpu.VMEM)] * (1 if n_out == 2 else 0)
        + [pl.BlockSpec(memory_space=pl.ANY)] * (n_out - (1 if n_out == 2 else 0)),
        scratch_shapes=[
            pltpu.VMEM((_NBUF, _CROWS, _D), jnp.float32),
            pltpu.VMEM((1, _D), jnp.float32),
            pltpu.SemaphoreType.DMA((_NBUF,)),
            pltpu.SemaphoreType.DMA((_NBUF,)),
            pltpu.SemaphoreType.DMA,
        ],
    )
    out_shape = ([jax.ShapeDtypeStruct((16, 1, _D), jnp.float32)] if n_out == 2
                 else []) + [jax.ShapeDtypeStruct((_R, _D), jnp.float32)]
    return pl.pallas_call(
        body, grid_spec=grid_spec, out_shape=out_shape,
        input_output_aliases=aliases)(*operands)


def _sc_tail_copy(v_flat):
    rows_per_w = _SC_TAIL // 32
    nchunks = rows_per_w // _SC_CHUNK
    head = _R - _SC_TAIL
    mesh = plsc.VectorSubcoreMesh(core_axis_name="c", subcore_axis_name="s")

    @functools.partial(
        pl.kernel,
        out_type=jax.ShapeDtypeStruct((_R, _D), jnp.float32),
        mesh=mesh,
        scratch_types=[
            pltpu.VMEM((_SC_NBUF, _SC_CHUNK, _D), jnp.float32),
            pltpu.SemaphoreType.DMA((_SC_NBUF,)),
            pltpu.SemaphoreType.DMA((_SC_NBUF,)),
        ],
    )
    def sc_kernel(v_hbm, out_hbm, bufs, in_sems, out_sems):
        wid = lax.axis_index("s") * 2 + lax.axis_index("c")
        base = head + wid * rows_per_w
        ins, outs = [], []
        for j in range(min(_SC_NBUF, nchunks)):
            c = pltpu.make_async_copy(
                v_hbm.at[pl.ds(base + j * _SC_CHUNK, _SC_CHUNK)],
                bufs.at[j], in_sems.at[j])
            c.start()
            ins.append(c)
        for i in range(nchunks):
            s = i % _SC_NBUF
            ins[i].wait()
            c = pltpu.make_async_copy(
                bufs.at[s], out_hbm.at[pl.ds(base + i * _SC_CHUNK, _SC_CHUNK)],
                out_sems.at[s])
            c.start()
            outs.append(c)
            ni = i + _SC_NBUF
            if ni < nchunks:
                outs[i].wait()
                c = pltpu.make_async_copy(
                    v_hbm.at[pl.ds(base + ni * _SC_CHUNK, _SC_CHUNK)],
                    bufs.at[s], in_sems.at[s])
                c.start()
                ins.append(c)
        for i in range(max(nchunks - _SC_NBUF, 0), nchunks):
            outs[i].wait()

    return sc_kernel(v_flat)


def kernel(x, k_cache, v_cache, cache_pos):
    B, S, D = k_cache.shape
    pos = jnp.asarray(cache_pos, jnp.int32).reshape(1)
    kf = k_cache.reshape(B * S, D)
    vf = v_cache.reshape(B * S, D)

    ox, ok = _tc_call(_tc1_body, 2, 2, (pos, x, kf), {})
    ovp = _sc_tail_copy(vf)
    ov = _tc_call(_tc2_body, 2, 1, (pos, vf, ovp), {2: 0})[0]

    return (ox, ok.reshape(B, S, D), ov.reshape(B, S, D))


# speedup vs baseline: 1.0263x; 1.0263x over previous
import functools

import jax
import jax.numpy as jnp
from jax import lax
from jax.experimental import pallas as pl
from jax.experimental.pallas import tpu as pltpu
from jax.experimental.pallas import tpu_sc as plsc

_R = 32768
_D = 1024
_CROWS = 2048
_NBUF = 4
_XW = 512


def _tc_body(pos_ref, k_hbm, v_hbm, ok_hbm, ov_hbm,
             bufs, row_buf, in_sems, out_sems, row_sems):
    pos = pos_ref[0]
    nper = _R // _CROWS

    def src(i):
        arr, j = (k_hbm, i // 2) if i % 2 == 0 else (v_hbm, i // 2)
        return arr.at[pl.ds(j * _CROWS, _CROWS)]

    def dst(i):
        arr, j = (ok_hbm, i // 2) if i % 2 == 0 else (ov_hbm, i // 2)
        return arr.at[pl.ds(j * _CROWS, _CROWS)]

    n = 2 * nper
    ins, outs = [], []
    for j in range(_NBUF):
        c = pltpu.make_async_copy(src(j), bufs.at[j], in_sems.at[j])
        c.start()
        ins.append(c)

    row_buf[0, :] = jnp.full((_D,), 100.0, jnp.float32)
    row_buf[1, :] = jnp.full((_D,), 200.0, jnp.float32)

    for i in range(n):
        s = i % _NBUF
        ins[i].wait()
        c = pltpu.make_async_copy(bufs.at[s], dst(i), out_sems.at[s])
        c.start()
        outs.append(c)
        ni = i + _NBUF
        if ni < n:
            outs[i].wait()
            c = pltpu.make_async_copy(src(ni), bufs.at[s], in_sems.at[s])
            c.start()
            ins.append(c)
    for i in range(max(n - _NBUF, 0), n):
        outs[i].wait()

    ck = pltpu.make_async_copy(
        row_buf.at[pl.ds(0, 1)], ok_hbm.at[pl.ds(2048 + pos, 1)], row_sems.at[0])
    cv = pltpu.make_async_copy(
        row_buf.at[pl.ds(1, 1)], ov_hbm.at[pl.ds(10245 + pos, 1)], row_sems.at[1])
    ck.start()
    cv.start()
    ck.wait()
    cv.wait()


def _tc_copy(pos, kf, vf):
    grid_spec = pltpu.PrefetchScalarGridSpec(
        num_scalar_prefetch=1,
        grid=(),
        in_specs=[pl.BlockSpec(memory_space=pl.ANY)] * 2,
        out_specs=[pl.BlockSpec(memory_space=pl.ANY)] * 2,
        scratch_shapes=[
            pltpu.VMEM((_NBUF, _CROWS, _D), jnp.float32),
            pltpu.VMEM((2, _D), jnp.float32),
            pltpu.SemaphoreType.DMA((_NBUF,)),
            pltpu.SemaphoreType.DMA((_NBUF,)),
            pltpu.SemaphoreType.DMA((2,)),
        ],
    )
    out_shape = [
        jax.ShapeDtypeStruct((_R, _D), jnp.float32),
        jax.ShapeDtypeStruct((_R, _D), jnp.float32),
    ]
    return pl.pallas_call(
        _tc_body, grid_spec=grid_spec, out_shape=out_shape)(pos, kf, vf)


def _sc_square(x_flat):
    n = x_flat.shape[0]
    mesh = plsc.VectorSubcoreMesh(core_axis_name="c", subcore_axis_name="s")

    @functools.partial(
        pl.kernel,
        out_type=jax.ShapeDtypeStruct((n,), jnp.float32),
        mesh=mesh,
        scratch_types=[
            pltpu.VMEM((_XW,), jnp.float32),
        ],
    )
    def sc_kernel(x_hbm, out_hbm, buf):
        wid = lax.axis_index("s") * 2 + lax.axis_index("c")
        base = wid * _XW
        pltpu.sync_copy(x_hbm.at[pl.ds(base, _XW)], buf)
        for j in range(_XW // 16):
            v = buf[pl.ds(j * 16, 16)]
            buf[pl.ds(j * 16, 16)] = v * v
        pltpu.sync_copy(buf, out_hbm.at[pl.ds(base, _XW)])

    return sc_kernel(x_flat)


def kernel(x, k_cache, v_cache, cache_pos):
    B, S, D = k_cache.shape
    pos = jnp.asarray(cache_pos, jnp.int32).reshape(1)
    kf = k_cache.reshape(B * S, D)
    vf = v_cache.reshape(B * S, D)

    ox = _sc_square(x.reshape(B * D))
    ok, ov = _tc_copy(pos, kf, vf)

    return (ox.reshape(B, 1, D),
            ok.reshape(B, S, D),
            ov.reshape(B, S, D))


# speedup vs baseline: 1.1289x; 1.1000x over previous
import jax
import jax.numpy as jnp
from jax.experimental import pallas as pl
from jax.experimental.pallas import tpu as pltpu

_R = 32768
_D = 1024
_CROWS = 4096
_NBUF = 3


def _body(pos_ref, x_ref, k_hbm, v_hbm, ox_ref, ok_hbm, ov_hbm,
          bufs, row_buf, in_sems, out_sems, row_sems):
    pos = pos_ref[0]
    nper = _R // _CROWS
    n = 2 * nper

    def src(i):
        arr, j = (k_hbm, i // 2) if i % 2 == 0 else (v_hbm, i // 2)
        return arr.at[pl.ds(j * _CROWS, _CROWS)]

    def dst(i):
        arr, j = (ok_hbm, i // 2) if i % 2 == 0 else (ov_hbm, i // 2)
        return arr.at[pl.ds(j * _CROWS, _CROWS)]

    ready = max(2 * (4095 // _CROWS), 2 * (12292 // _CROWS) + 1)

    ins, outs = [], []
    for j in range(_NBUF):
        c = pltpu.make_async_copy(src(j), bufs.at[j], in_sems.at[j])
        c.start()
        ins.append(c)

    ox_ref[...] = x_ref[...] * x_ref[...]
    row_buf[0, :] = jnp.full((_D,), 100.0, jnp.float32)
    row_buf[1, :] = jnp.full((_D,), 200.0, jnp.float32)

    rows_started = []
    for i in range(n):
        s = i % _NBUF
        ins[i].wait()
        c = pltpu.make_async_copy(bufs.at[s], dst(i), out_sems.at[s])
        c.start()
        outs.append(c)
        ni = i + _NBUF
        if ni < n:
            outs[i].wait()
            c = pltpu.make_async_copy(src(ni), bufs.at[s], in_sems.at[s])
            c.start()
            ins.append(c)
            if i == ready:
                ck = pltpu.make_async_copy(
                    row_buf.at[pl.ds(0, 1)],
                    ok_hbm.at[pl.ds(2048 + pos, 1)], row_sems.at[0])
                cv = pltpu.make_async_copy(
                    row_buf.at[pl.ds(1, 1)],
                    ov_hbm.at[pl.ds(10245 + pos, 1)], row_sems.at[1])
                ck.start()
                cv.start()
                rows_started = [ck, cv]
    for i in range(max(n - _NBUF, 0), n):
        outs[i].wait()
    for c in rows_started:
        c.wait()


def kernel(x, k_cache, v_cache, cache_pos):
    B, S, D = k_cache.shape
    pos = jnp.asarray(cache_pos, jnp.int32).reshape(1)
    kf = k_cache.reshape(B * S, D)
    vf = v_cache.reshape(B * S, D)
    grid_spec = pltpu.PrefetchScalarGridSpec(
        num_scalar_prefetch=1,
        grid=(),
        in_specs=[
            pl.BlockSpec(memory_space=pltpu.VMEM),
            pl.BlockSpec(memory_space=pl.ANY),
            pl.BlockSpec(memory_space=pl.ANY),
        ],
        out_specs=[
            pl.BlockSpec(memory_space=pltpu.VMEM),
            pl.BlockSpec(memory_space=pl.ANY),
            pl.BlockSpec(memory_space=pl.ANY),
        ],
        scratch_shapes=[
            pltpu.VMEM((_NBUF, _CROWS, _D), jnp.float32),
            pltpu.VMEM((2, _D), jnp.float32),
            pltpu.SemaphoreType.DMA((_NBUF,)),
            pltpu.SemaphoreType.DMA((_NBUF,)),
            pltpu.SemaphoreType.DMA((2,)),
        ],
    )
    out_shape = [
        jax.ShapeDtypeStruct(x.shape, x.dtype),
        jax.ShapeDtypeStruct((B * S, D), jnp.float32),
        jax.ShapeDtypeStruct((B * S, D), jnp.float32),
    ]
    ox, ok, ov = pl.pallas_call(
        _body, grid_spec=grid_spec, out_shape=out_shape)(pos, x, kf, vf)
    return (ox, ok.reshape(B, S, D), ov.reshape(B, S, D))
